# speedup vs baseline: 1.0353x; 1.0353x over previous
import functools

import jax
import jax.numpy as jnp
from jax import lax
from jax.experimental import pallas as pl
from jax.experimental.pallas import tpu as pltpu
from jax.experimental.pallas import tpu_sc as plsc

_NUM = 8192
_DIM = 128
_N = 8192
_BZ = 512
_CK = 1024
_NCHUNK = _NUM // _CK

_NC = 2
_NS = 16
_NW = _NC * _NS
_BPW = _N // _NW
_IDXC = 128


def _argmin_body(z_ref, zn_ref, w_ref, wn_ref, ind_ref):
    zb = z_ref[...]
    zn = zn_ref[...]

    def chunk(c, carry):
        bval, bidx = carry
        wblk = w_ref[pl.ds(c * _CK, _CK), :]
        wn = wn_ref[0, pl.ds(c * _CK, _CK)]
        mm = lax.dot_general(zb, wblk, (((1,), (1,)), ((), ())),
                             preferred_element_type=jnp.float32)
        dis = (zn + wn[None, :]) - 2.0 * mm
        lmin = jnp.min(dis, axis=1)
        io = lax.broadcasted_iota(jnp.int32, (_BZ, _CK), 1) + c * _CK
        lidx = jnp.min(jnp.where(dis == lmin[:, None], io,
                                 jnp.int32(2147483647)), axis=1)
        take = lmin < bval
        return (jnp.where(take, lmin, bval), jnp.where(take, lidx, bidx))

    init = (jnp.full((_BZ,), jnp.inf, jnp.float32),
            jnp.zeros((_BZ,), jnp.int32))
    _, bidx = lax.fori_loop(0, _NCHUNK, chunk, init)
    ind_ref[0, 0, :] = bidx


def _argmin_call(z_flat, znorm, w, wnorm2d):
    return pl.pallas_call(
        _argmin_body,
        grid=(_N // _BZ,),
        in_specs=[
            pl.BlockSpec((_BZ, _DIM), lambda i: (i, 0)),
            pl.BlockSpec((_BZ, 1), lambda i: (i, 0)),
            pl.BlockSpec((_NUM, _DIM), lambda i: (0, 0)),
            pl.BlockSpec((1, _NUM), lambda i: (0, 0)),
        ],
        out_specs=pl.BlockSpec((1, 1, _BZ), lambda i: (i, 0, 0)),
        out_shape=jax.ShapeDtypeStruct((_N // _BZ, 1, _BZ), jnp.int32),
    )(z_flat, znorm, w, wnorm2d)


_SC_MESH = plsc.VectorSubcoreMesh(core_axis_name="c", subcore_axis_name="s")


@functools.partial(
    pl.kernel,
    mesh=_SC_MESH,
    out_type=(
        jax.ShapeDtypeStruct((_N, _DIM), jnp.float32),
        jax.ShapeDtypeStruct((_NW, 16), jnp.float32),
    ),
    scratch_types=[
        pltpu.VMEM((_BPW // _IDXC, _IDXC), jnp.int32),
        pltpu.VMEM((_BPW, _DIM), jnp.float32),
        pltpu.VMEM((_BPW, _DIM), jnp.float32),
        pltpu.VMEM((16,), jnp.float32),
        pltpu.SemaphoreType.DMA,
    ],
)
def _gather_loss(w_hbm, ind_hbm, z_hbm, zq_hbm, part_hbm,
                 idx_v, rows_v, z_v, acc_v, sem):
    wid = lax.axis_index("s") * _NC + lax.axis_index("c")
    nidx = _BPW // _IDXC
    base = wid * _BPW
    pltpu.sync_copy(ind_hbm.at[pl.ds(wid * nidx, nidx), :], idx_v)
    cps = []
    for j in range(nidx):
        cps.append(pltpu.async_copy(
            w_hbm.at[idx_v.at[j]],
            rows_v.at[pl.ds(j * _IDXC, _IDXC), :], sem))
    pltpu.sync_copy(z_hbm.at[pl.ds(base, _BPW), :], z_v)
    for cp in cps:
        cp.wait()

    def rbody(r, acc):
        def cbody(c, acc2):
            d = rows_v[r, pl.ds(c * 16, 16)] - z_v[r, pl.ds(c * 16, 16)]
            return acc2 + d * d
        return lax.fori_loop(0, _DIM // 16, cbody, acc)

    acc = lax.fori_loop(0, _BPW, rbody, jnp.zeros((16,), jnp.float32))
    acc_v[...] = acc
    pltpu.sync_copy(rows_v, zq_hbm.at[pl.ds(base, _BPW), :])
    pltpu.sync_copy(acc_v, part_hbm.at[wid])


def kernel(batch, vq_weight):
    b, c, h, w = batch.shape
    z = batch
    z_flat = jnp.transpose(z, (0, 2, 3, 1)).reshape(-1, _DIM)
    znorm = jnp.sum(z_flat ** 2, axis=1, keepdims=True)
    wnorm = jnp.sum(vq_weight ** 2, axis=1)

    ind3 = _argmin_call(z_flat, znorm, vq_weight, wnorm.reshape(1, _NUM))
    ind64 = ind3.reshape(_N // _IDXC, _IDXC)

    zq_flat, partials = _gather_loss(vq_weight, ind64, z_flat)

    z_q = jnp.transpose(zq_flat.reshape(b, h, w, c), (0, 3, 1, 2))
    out = z + lax.stop_gradient(z_q - z)
    loss = 52.0 * (jnp.sum(partials) / (b * c * h * w))
    return (out, loss)


# speedup vs baseline: 1.1857x; 1.1453x over previous
import functools

import jax
import jax.numpy as jnp
from jax import lax
from jax.experimental import pallas as pl
from jax.experimental.pallas import tpu as pltpu
from jax.experimental.pallas import tpu_sc as plsc

_NUM = 8192
_DIM = 128
_N = 8192
_BZ = 512
_CK = 1024
_NCHUNK = _NUM // _CK

_NC = 2
_NS = 16
_NW = _NC * _NS
_BPW = _N // _NW
_IDXC = 128


def _argmin_body(z_ref, zn_ref, w_ref, wn_ref, ind_ref):
    zb = z_ref[...]
    zn = zn_ref[...]

    def chunk(c, carry):
        bval, bidx = carry
        wblk = w_ref[pl.ds(c * _CK, _CK), :]
        wn = wn_ref[0, pl.ds(c * _CK, _CK)]
        mm = lax.dot_general(zb, wblk, (((1,), (1,)), ((), ())),
                             preferred_element_type=jnp.float32)
        dis = (zn + wn[None, :]) - 2.0 * mm
        lmin = jnp.min(dis, axis=1)
        io = lax.broadcasted_iota(jnp.int32, (_BZ, _CK), 1) + c * _CK
        lidx = jnp.min(jnp.where(dis == lmin[:, None], io,
                                 jnp.int32(2147483647)), axis=1)
        take = lmin < bval
        return (jnp.where(take, lmin, bval), jnp.where(take, lidx, bidx))

    carry = (jnp.full((_BZ,), jnp.inf, jnp.float32),
             jnp.zeros((_BZ,), jnp.int32))
    for c in range(_NCHUNK):
        carry = chunk(c, carry)
    ind_ref[0, 0, :] = carry[1]


def _argmin_call(z_flat, znorm, w, wnorm2d):
    return pl.pallas_call(
        _argmin_body,
        grid=(_N // _BZ,),
        in_specs=[
            pl.BlockSpec((_BZ, _DIM), lambda i: (i, 0)),
            pl.BlockSpec((_BZ, 1), lambda i: (i, 0)),
            pl.BlockSpec((_NUM, _DIM), lambda i: (0, 0)),
            pl.BlockSpec((1, _NUM), lambda i: (0, 0)),
        ],
        out_specs=pl.BlockSpec((1, 1, _BZ), lambda i: (i, 0, 0)),
        out_shape=jax.ShapeDtypeStruct((_N // _BZ, 1, _BZ), jnp.int32),
    )(z_flat, znorm, w, wnorm2d)


_SC_MESH = plsc.VectorSubcoreMesh(core_axis_name="c", subcore_axis_name="s")


@functools.partial(
    pl.kernel,
    mesh=_SC_MESH,
    out_type=(
        jax.ShapeDtypeStruct((_N, _DIM), jnp.float32),
        jax.ShapeDtypeStruct((_NW, 16), jnp.float32),
    ),
    scratch_types=[
        pltpu.VMEM((_BPW // _IDXC, _IDXC), jnp.int32),
        pltpu.VMEM((_BPW, _DIM), jnp.float32),
        pltpu.VMEM((_BPW, _DIM), jnp.float32),
        pltpu.VMEM((16,), jnp.float32),
        pltpu.SemaphoreType.DMA,
    ],
)
def _gather_loss(w_hbm, ind_hbm, z_hbm, zq_hbm, part_hbm,
                 idx_v, rows_v, z_v, acc_v, sem):
    wid = lax.axis_index("s") * _NC + lax.axis_index("c")
    nidx = _BPW // _IDXC
    base = wid * _BPW
    pltpu.sync_copy(ind_hbm.at[pl.ds(wid * nidx, nidx), :], idx_v)
    cps = []
    for j in range(nidx):
        cps.append(pltpu.async_copy(
            w_hbm.at[idx_v.at[j]],
            rows_v.at[pl.ds(j * _IDXC, _IDXC), :], sem))
    pltpu.sync_copy(z_hbm.at[pl.ds(base, _BPW), :], z_v)
    for cp in cps:
        cp.wait()

    def rbody(r, acc):
        def cbody(c, acc2):
            d = rows_v[r, pl.ds(c * 16, 16)] - z_v[r, pl.ds(c * 16, 16)]
            return acc2 + d * d
        return lax.fori_loop(0, _DIM // 16, cbody, acc)

    acc = lax.fori_loop(0, _BPW, rbody, jnp.zeros((16,), jnp.float32))
    acc_v[...] = acc
    pltpu.sync_copy(rows_v, zq_hbm.at[pl.ds(base, _BPW), :])
    pltpu.sync_copy(acc_v, part_hbm.at[wid])


def kernel(batch, vq_weight):
    b, c, h, w = batch.shape
    z = batch
    z_flat = jnp.transpose(z, (0, 2, 3, 1)).reshape(-1, _DIM)
    znorm = jnp.sum(z_flat ** 2, axis=1, keepdims=True)
    wnorm = jnp.sum(vq_weight ** 2, axis=1)

    ind3 = _argmin_call(z_flat, znorm, vq_weight, wnorm.reshape(1, _NUM))
    ind64 = ind3.reshape(_N // _IDXC, _IDXC)

    zq_flat, partials = _gather_loss(vq_weight, ind64, z_flat)

    z_q = jnp.transpose(zq_flat.reshape(b, h, w, c), (0, 3, 1, 2))
    out = z + lax.stop_gradient(z_q - z)
    loss = 52.0 * (jnp.sum(partials) / (b * c * h * w))
    return (out, loss)
